# Initial kernel scaffold; baseline (speedup 1.0000x reference)
import functools

import jax
import jax.numpy as jnp
import numpy as np
from jax import lax
from jax.experimental import pallas as pl
from jax.experimental.pallas import tpu as pltpu
from jax.experimental.pallas import tpu_sc as plsc

B = 16384
V = 1000000
D = 16
NOH = 26
L = 20
NC = 13
F = 48
NW = 32
IDX_COLS = 128
N_IDX_ROWS = B * F // IDX_COLS
ROWS_PER_W = N_IDX_ROWS // NW
G = 8
N_GROUPS = ROWS_PER_W // G


def _sc_gather(idx2d, deep_table, wide_table):
    mesh = plsc.VectorSubcoreMesh(core_axis_name="c", subcore_axis_name="s")

    @functools.partial(
        pl.kernel,
        out_type=(
            jax.ShapeDtypeStruct((B * F, D), jnp.float32),
            jax.ShapeDtypeStruct((B * F, 1), jnp.float32),
        ),
        mesh=mesh,
        scratch_types=[
            pltpu.VMEM((G, IDX_COLS), jnp.int32),
            pltpu.VMEM((G * IDX_COLS, D), jnp.float32),
            pltpu.VMEM((G * IDX_COLS, 1), jnp.float32),
            pltpu.SemaphoreType.DMA,
            pltpu.SemaphoreType.DMA,
        ],
    )
    def k(idx_hbm, deep_hbm, wide_hbm, u_hbm, wv_hbm, idx_v, deep_v, wide_v,
          sem_d, sem_w):
        wid = lax.axis_index("s") * 2 + lax.axis_index("c")

        def body(g, _):
            rowbase = wid * ROWS_PER_W + g * G
            pltpu.sync_copy(idx_hbm.at[pl.ds(rowbase, G)], idx_v)
            for j in range(G):
                pltpu.async_copy(
                    deep_hbm.at[idx_v.at[j]],
                    deep_v.at[pl.ds(j * IDX_COLS, IDX_COLS)], sem_d)
                pltpu.async_copy(
                    wide_hbm.at[idx_v.at[j]],
                    wide_v.at[pl.ds(j * IDX_COLS, IDX_COLS)], sem_w)
            for j in range(G):
                pltpu.make_async_copy(
                    deep_hbm.at[idx_v.at[j]],
                    deep_v.at[pl.ds(j * IDX_COLS, IDX_COLS)], sem_d).wait()
                pltpu.make_async_copy(
                    wide_hbm.at[idx_v.at[j]],
                    wide_v.at[pl.ds(j * IDX_COLS, IDX_COLS)], sem_w).wait()
            flatbase = rowbase * IDX_COLS
            pltpu.sync_copy(deep_v, u_hbm.at[pl.ds(flatbase, G * IDX_COLS)])
            pltpu.sync_copy(wide_v, wv_hbm.at[pl.ds(flatbase, G * IDX_COLS)])
            return 0

        lax.fori_loop(0, N_GROUPS, body, 0)

    return k(idx2d, deep_table, wide_table)


def _leaky(v):
    return jnp.where(v >= 0, v, 0.01 * v)


def _mlp_body(u_ref, wgt_ref, wv_ref, ctns_ref, e_ref, w2big_ref, w2c_ref,
              b2_ref, w3_ref, b3_ref, w4_ref, b4_ref, out_ref):
    wgt = wgt_ref[...]
    wt = jax.lax.dot_general(wgt, e_ref[...], (((1,), (0,)), ((), ())),
                             preferred_element_type=jnp.float32)
    x = u_ref[...] * wt
    h = jax.lax.dot_general(x, w2big_ref[...], (((1,), (1,)), ((), ())),
                            preferred_element_type=jnp.float32)
    h += jax.lax.dot_general(ctns_ref[...], w2c_ref[...],
                             (((1,), (1,)), ((), ())),
                             preferred_element_type=jnp.float32)
    h = _leaky(h + b2_ref[...])
    h = jax.lax.dot_general(h, w3_ref[...], (((1,), (1,)), ((), ())),
                            preferred_element_type=jnp.float32)
    h = _leaky(h + b3_ref[...])
    h = jax.lax.dot_general(h, w4_ref[...], (((1,), (1,)), ((), ())),
                            preferred_element_type=jnp.float32)
    wide = 16.0 * jnp.sum(wv_ref[...] * wgt, axis=1, keepdims=True)
    out_ref[...] = jax.nn.sigmoid(h + b4_ref[...] + wide)


def _tc_mlp(u, wgt, widev, ctns, e_mat, w2big, w2c, b2, w3, b3, w4, b4):
    bt = 1024
    grid = (B // bt,)
    full = lambda shape: pl.BlockSpec(shape, lambda i: (0, 0))
    return pl.pallas_call(
        _mlp_body,
        grid=grid,
        in_specs=[
            pl.BlockSpec((bt, F * D), lambda i: (i, 0)),
            pl.BlockSpec((bt, F), lambda i: (i, 0)),
            pl.BlockSpec((bt, F), lambda i: (i, 0)),
            pl.BlockSpec((bt, NC), lambda i: (i, 0)),
            full((F, F * D)),
            full((256, F * D)),
            full((256, NC)),
            full((1, 256)),
            full((128, 256)),
            full((1, 128)),
            full((1, 128)),
            full((1, 1)),
        ],
        out_specs=pl.BlockSpec((bt, 1), lambda i: (i, 0)),
        out_shape=jax.ShapeDtypeStruct((B, 1), jnp.float32),
    )(u, wgt, widev, ctns, e_mat, w2big, w2c, b2, w3, b3, w4, b4)


_E_MAT = np.kron(np.eye(F, dtype=np.float32), np.ones((1, D), np.float32))


def kernel(onehot_i, onehot_x, multihot_list, ctns, wide_table, deep_table,
           w2, b2, w3, b3, w4, b4):
    mh_i = multihot_list[0, 0]
    mh_x = multihot_list[0, 1].astype(jnp.float32)
    zeros_i = jnp.zeros((B, F - NOH - L), jnp.int32)
    zeros_x = jnp.zeros((B, F - NOH - L), jnp.float32)
    idx = jnp.concatenate([onehot_i, mh_i, zeros_i], axis=1)
    wgt = jnp.concatenate([onehot_x, mh_x, zeros_x], axis=1)
    idx2d = idx.reshape(N_IDX_ROWS, IDX_COLS)

    u, widev = _sc_gather(idx2d, deep_table, wide_table)
    u = u.reshape(B, F * D)
    widev = widev.reshape(B, F)

    w2a = w2[:, :NOH * D]
    w2b = w2[:, NOH * D:NOH * D + D]
    w2c = w2[:, NOH * D + D:]
    w2big = jnp.concatenate(
        [w2a] + [w2b] * L + [jnp.zeros((256, 2 * D), jnp.float32)], axis=1)

    out = _tc_mlp(u, wgt, widev, ctns, jnp.asarray(_E_MAT), w2big, w2c,
                  b2.reshape(1, 256), w3, b3.reshape(1, 128),
                  w4, b4.reshape(1, 1))
    return out.reshape(B)


# speedup vs baseline: 1.1332x; 1.1332x over previous
import functools

import jax
import jax.numpy as jnp
import numpy as np
from jax import lax
from jax.experimental import pallas as pl
from jax.experimental.pallas import tpu as pltpu
from jax.experimental.pallas import tpu_sc as plsc

B = 16384
V = 1000000
D = 16
NOH = 26
L = 20
NC = 13
F = 48
NW = 32
IDX_COLS = 128
N_IDX_ROWS = B * F // IDX_COLS
ROWS_PER_W = N_IDX_ROWS // NW
G = 8
N_GROUPS = ROWS_PER_W // G


def _sc_gather(idx_flat, ridx_flat, lane_flat, deep_table, wide16):
    mesh = plsc.VectorSubcoreMesh(core_axis_name="c", subcore_axis_name="s")
    GN = G * IDX_COLS

    @functools.partial(
        pl.kernel,
        out_type=(
            jax.ShapeDtypeStruct((B * F, D), jnp.float32),
            jax.ShapeDtypeStruct((B * F,), jnp.float32),
        ),
        mesh=mesh,
        scratch_types=[
            [pltpu.VMEM((IDX_COLS,), jnp.int32) for _ in range(G)],
            [pltpu.VMEM((IDX_COLS,), jnp.int32) for _ in range(G)],
            pltpu.VMEM((GN,), jnp.int32),
            pltpu.VMEM((GN, D), jnp.float32),
            pltpu.VMEM((GN, D), jnp.float32),
            pltpu.VMEM((GN,), jnp.float32),
            pltpu.SemaphoreType.DMA,
            pltpu.SemaphoreType.DMA,
        ],
        compiler_params=pltpu.CompilerParams(use_tc_tiling_on_sc=False,
                                             needs_layout_passes=False),
    )
    def k(idx_hbm, ridx_hbm, lane_hbm, deep_hbm, wide_hbm, u_hbm, wv_hbm,
          idx_vs, ridx_vs, lane_v, deep_v, wrow_v, wide_v, sem_d, sem_w):
        wid = lax.axis_index("s") * 2 + lax.axis_index("c")

        def body(g, _):
            flatbase = (wid * ROWS_PER_W + g * G) * IDX_COLS
            for j in range(G):
                pltpu.sync_copy(
                    idx_hbm.at[pl.ds(flatbase + j * IDX_COLS, IDX_COLS)],
                    idx_vs[j])
                pltpu.sync_copy(
                    ridx_hbm.at[pl.ds(flatbase + j * IDX_COLS, IDX_COLS)],
                    ridx_vs[j])
            pltpu.sync_copy(lane_hbm.at[pl.ds(flatbase, GN)], lane_v)
            for j in range(G):
                pltpu.async_copy(
                    deep_hbm.at[idx_vs[j]],
                    deep_v.at[pl.ds(j * IDX_COLS, IDX_COLS)], sem_d)
                pltpu.async_copy(
                    wide_hbm.at[ridx_vs[j]],
                    wrow_v.at[pl.ds(j * IDX_COLS, IDX_COLS)], sem_w)
            for j in range(G):
                pltpu.make_async_copy(
                    deep_hbm.at[idx_vs[j]],
                    deep_v.at[pl.ds(j * IDX_COLS, IDX_COLS)], sem_d).wait()
                pltpu.make_async_copy(
                    wide_hbm.at[ridx_vs[j]],
                    wrow_v.at[pl.ds(j * IDX_COLS, IDX_COLS)], sem_w).wait()
            for t in range(GN // 16):
                p = t * 16
                rowv = lax.broadcasted_iota(jnp.int32, (16,), 0) + p
                lanev = lane_v[pl.ds(p, 16)]
                wide_v[pl.ds(p, 16)] = plsc.load_gather(
                    wrow_v, [rowv, lanev])
            pltpu.sync_copy(deep_v, u_hbm.at[pl.ds(flatbase, GN)])
            pltpu.sync_copy(wide_v, wv_hbm.at[pl.ds(flatbase, GN)])
            return 0

        lax.fori_loop(0, N_GROUPS, body, 0)

    return k(idx_flat, ridx_flat, lane_flat, deep_table, wide16)


def _leaky(v):
    return jnp.where(v >= 0, v, 0.01 * v)


def _mlp_body(u_ref, wgt_ref, wv_ref, ctns_ref, e_ref, w2big_ref, w2c_ref,
              b2_ref, w3_ref, b3_ref, w4t_ref, b4_ref, out_ref):
    wgt = wgt_ref[...]
    wt = jax.lax.dot_general(wgt, e_ref[...], (((1,), (0,)), ((), ())),
                             preferred_element_type=jnp.float32,
                            precision=jax.lax.Precision.HIGHEST)
    x = u_ref[...] * wt
    h = jax.lax.dot_general(x, w2big_ref[...], (((1,), (1,)), ((), ())),
                            preferred_element_type=jnp.float32,
                            precision=jax.lax.Precision.HIGHEST)
    h += jax.lax.dot_general(ctns_ref[...], w2c_ref[...],
                             (((1,), (1,)), ((), ())),
                             preferred_element_type=jnp.float32,
                            precision=jax.lax.Precision.HIGHEST)
    h = _leaky(h + b2_ref[...])
    h = jax.lax.dot_general(h, w3_ref[...], (((1,), (1,)), ((), ())),
                            preferred_element_type=jnp.float32,
                            precision=jax.lax.Precision.HIGHEST)
    h = _leaky(h + b3_ref[...])
    h = jax.lax.dot_general(h, w4t_ref[...], (((1,), (0,)), ((), ())),
                            preferred_element_type=jnp.float32,
                            precision=jax.lax.Precision.HIGHEST)
    wide = 16.0 * jnp.sum(wv_ref[...] * wgt, axis=1, keepdims=True)
    out_ref[...] = jax.nn.sigmoid(h + b4_ref[0, 0] + wide)


def _tc_mlp(u, wgt, widev, ctns, e_mat, w2big, w2c, b2, w3, b3, w4, b4):
    bt = 1024
    grid = (B // bt,)
    full = lambda shape: pl.BlockSpec(shape, lambda i: (0, 0))
    return pl.pallas_call(
        _mlp_body,
        grid=grid,
        in_specs=[
            pl.BlockSpec((bt, F * D), lambda i: (i, 0)),
            pl.BlockSpec((bt, F), lambda i: (i, 0)),
            pl.BlockSpec((bt, F), lambda i: (i, 0)),
            pl.BlockSpec((bt, NC), lambda i: (i, 0)),
            full((F, F * D)),
            full((256, F * D)),
            full((256, NC)),
            full((1, 256)),
            full((128, 256)),
            full((1, 128)),
            full((128, 1)),
            pl.BlockSpec(memory_space=pltpu.SMEM),
        ],
        out_specs=pl.BlockSpec((bt, 1), lambda i: (i, 0)),
        out_shape=jax.ShapeDtypeStruct((B, 1), jnp.float32),
    )(u, wgt, widev, ctns, e_mat, w2big, w2c, b2, w3, b3, w4, b4)


_E_MAT = np.kron(np.eye(F, dtype=np.float32), np.ones((1, D), np.float32))


def kernel(onehot_i, onehot_x, multihot_list, ctns, wide_table, deep_table,
           w2, b2, w3, b3, w4, b4):
    mh_i = multihot_list[0, 0]
    mh_x = multihot_list[0, 1].astype(jnp.float32)
    zeros_i = jnp.zeros((B, F - NOH - L), jnp.int32)
    zeros_x = jnp.zeros((B, F - NOH - L), jnp.float32)
    idx = jnp.concatenate([onehot_i, mh_i, zeros_i], axis=1)
    wgt = jnp.concatenate([onehot_x, mh_x, zeros_x], axis=1)
    idx_flat = idx.reshape(-1)
    u, widev = _sc_gather(idx_flat, idx_flat >> 4, idx_flat & 15,
                          deep_table, wide_table.reshape(V // 16, 16))
    u = u.reshape(B, F * D)
    widev = widev.reshape(B, F)

    w2a = w2[:, :NOH * D]
    w2b = w2[:, NOH * D:NOH * D + D]
    w2c = w2[:, NOH * D + D:]
    w2big = jnp.concatenate(
        [w2a] + [w2b] * L + [jnp.zeros((256, 2 * D), jnp.float32)], axis=1)

    out = _tc_mlp(u, wgt, widev, ctns, jnp.asarray(_E_MAT), w2big, w2c,
                  b2.reshape(1, 256), w3, b3.reshape(1, 128),
                  w4.reshape(128, 1), b4.reshape(1, 1))
    return out.reshape(B)
